# baseline (speedup 1.0000x reference)
import jax
import jax.numpy as jnp
from jax.experimental import pallas as pl

OUT_D = 4096
ROWS = 5
BB = 32


def _body(floats_ref, out_ref):
    n = BB * ROWS
    r = jax.lax.broadcasted_iota(jnp.int32, (n, 1), 0) % ROWS
    begin = floats_ref[:, 0:1]
    end = floats_ref[:, 1:2]
    step = (end - begin) * (1.0 / 4095.0)

    is0 = r == 0
    is1 = r == 1
    is4 = r == 4
    one = jnp.ones_like(begin)
    zero = jnp.zeros_like(begin)
    a = jnp.where(is0 | is1, one, jnp.where(is4, -one, zero))
    b = jnp.where(is0 | is4, zero, -one)
    c = jnp.where(is1, one, jnp.where(is0 | is4, -one, zero))
    d = jnp.where(is0, 2.0 * one, jnp.where(is1, -one, zero))

    A = a * step
    B = a * begin + b
    C = c * step
    D = c * begin + d

    i = jax.lax.broadcasted_iota(jnp.int32, (n, OUT_D), 1).astype(jnp.float32)
    out_ref[...] = jnp.maximum(0.0, jnp.minimum(A * i + B, C * i + D))


def kernel(floats):
    bsz = floats.shape[0]
    floats5 = jnp.repeat(floats, ROWS, axis=0)
    flat = pl.pallas_call(
        _body,
        grid=(bsz // BB,),
        in_specs=[pl.BlockSpec((BB * ROWS, 2), lambda i: (i, 0))],
        out_specs=pl.BlockSpec((BB * ROWS, OUT_D), lambda i: (i, 0)),
        out_shape=jax.ShapeDtypeStruct((bsz * ROWS, OUT_D), jnp.float32),
    )(floats5)
    return (flat.reshape(bsz, ROWS, OUT_D), jnp.ones((bsz, 1), jnp.float32))


# speedup vs baseline: 1.1690x; 1.1690x over previous
import jax
import jax.numpy as jnp
from jax.experimental import pallas as pl

OUT_D = 4096
ROWS = 5
BB = 8


def _body(floats_ref, out_ref):
    begin = floats_ref[:, 0:1].reshape(BB, 1, 1)
    end = floats_ref[:, 1:2].reshape(BB, 1, 1)
    step = (end - begin) * (1.0 / 4095.0)
    r = jax.lax.broadcasted_iota(jnp.int32, (1, ROWS, 1), 1)
    p = jnp.where(r == 0, 1.0,
                  jnp.where(r == 1, 2.0, jnp.where(r == 4, -1.0, 4.0)))
    off = begin - p
    i = jax.lax.broadcasted_iota(jnp.int32, (BB, ROWS, OUT_D), 2)
    q = i.astype(jnp.float32) * step + off
    out_ref[...] = jnp.maximum(0.0, 1.0 - jnp.abs(q))


def kernel(floats):
    bsz = floats.shape[0]
    mats = pl.pallas_call(
        _body,
        grid=(bsz // BB,),
        in_specs=[pl.BlockSpec((BB, 2), lambda i: (i, 0))],
        out_specs=pl.BlockSpec((BB, ROWS, OUT_D), lambda i: (i, 0, 0)),
        out_shape=jax.ShapeDtypeStruct((bsz, ROWS, OUT_D), jnp.float32),
    )(floats)
    return (mats, jnp.ones((bsz, 1), jnp.float32))


# speedup vs baseline: 1.6099x; 1.3772x over previous
import jax
import jax.numpy as jnp
from jax.experimental import pallas as pl

OUT_D = 4096
ROWS = 5
BB = 32


def _body(floats_ref, out_ref):
    begin = floats_ref[:, 0:1].reshape(BB, 1, 1)
    end = floats_ref[:, 1:2].reshape(BB, 1, 1)
    step = (end - begin) * (1.0 / 4095.0)
    r = jax.lax.broadcasted_iota(jnp.int32, (1, ROWS, 1), 1)
    p = jnp.where(r == 0, 1.0,
                  jnp.where(r == 1, 2.0, jnp.where(r == 4, -1.0, 4.0)))
    off = begin - p
    i = jax.lax.broadcasted_iota(jnp.int32, (BB, ROWS, OUT_D), 2)
    q = i.astype(jnp.float32) * step + off
    out_ref[...] = jnp.maximum(0.0, 1.0 - jnp.abs(q))


def kernel(floats):
    bsz = floats.shape[0]
    mats = pl.pallas_call(
        _body,
        grid=(bsz // BB,),
        in_specs=[pl.BlockSpec((BB, 2), lambda i: (i, 0))],
        out_specs=pl.BlockSpec((BB, ROWS, OUT_D), lambda i: (i, 0, 0)),
        out_shape=jax.ShapeDtypeStruct((bsz, ROWS, OUT_D), jnp.float32),
    )(floats)
    return (mats, jnp.ones((bsz, 1), jnp.float32))


# speedup vs baseline: 1.7312x; 1.0754x over previous
import jax
import jax.numpy as jnp
from jax.experimental import pallas as pl

OUT_D = 4096
ROWS = 5
BB = 64


def _body(floats_ref, out_ref):
    begin = floats_ref[:, 0:1].reshape(BB, 1, 1)
    end = floats_ref[:, 1:2].reshape(BB, 1, 1)
    step = (end - begin) * (1.0 / 4095.0)
    r = jax.lax.broadcasted_iota(jnp.int32, (1, ROWS, 1), 1)
    p = jnp.where(r == 0, 1.0,
                  jnp.where(r == 1, 2.0, jnp.where(r == 4, -1.0, 4.0)))
    off = begin - p
    i = jax.lax.broadcasted_iota(jnp.int32, (BB, ROWS, OUT_D), 2)
    q = i.astype(jnp.float32) * step + off
    out_ref[...] = jnp.maximum(0.0, 1.0 - jnp.abs(q))


def kernel(floats):
    bsz = floats.shape[0]
    mats = pl.pallas_call(
        _body,
        grid=(bsz // BB,),
        in_specs=[pl.BlockSpec((BB, 2), lambda i: (i, 0))],
        out_specs=pl.BlockSpec((BB, ROWS, OUT_D), lambda i: (i, 0, 0)),
        out_shape=jax.ShapeDtypeStruct((bsz, ROWS, OUT_D), jnp.float32),
    )(floats)
    return (mats, jnp.ones((bsz, 1), jnp.float32))
